# Initial kernel scaffold; baseline (speedup 1.0000x reference)
import jax
import jax.numpy as jnp
from jax.experimental import pallas as pl


def kernel(features, preds, current_alpha):
    raise NotImplementedError("write your pallas kernel here")


# speedup vs baseline: 6.7752x; 6.7752x over previous
import functools

import jax
import jax.numpy as jnp
from jax.experimental import pallas as pl

_B = 4096
_D = 1024
_C = 1000
_K = 32
_RB = 512


def _normalize_kernel(f_ref, out_ref):
    f = f_ref[...]
    n2 = jnp.sum(f * f, axis=1, keepdims=True)
    out_ref[...] = f * jax.lax.rsqrt(jnp.maximum(n2, 1e-24))


def _simtopk_kernel(fb_ref, fn_ref, tv_ref, ti_ref):
    sim = jax.lax.dot_general(
        fb_ref[...], fn_ref[...], (((1,), (1,)), ((), ())),
        preferred_element_type=jnp.float32)
    col = jax.lax.broadcasted_iota(jnp.int32, sim.shape, 1)
    vals = sim
    neg = jnp.float32(-3.0e38)
    for t in range(_K):
        m = jnp.max(vals, axis=1, keepdims=True)
        am = jnp.min(jnp.where(vals == m, col, _B), axis=1)
        tv_ref[:, t] = m[:, 0]
        ti_ref[:, t] = am
        vals = jnp.where(col == am[:, None], neg, vals)


def _scale_kernel(tv_ref, preds_ref, d_ref, ps_ref):
    rowsum = jnp.sum(tv_ref[...], axis=1, keepdims=True) + 1.0
    d = jax.lax.rsqrt(rowsum)
    d = jnp.where(jnp.isinf(d), 0.0, d)
    d_ref[...] = d
    ps_ref[...] = preds_ref[...] * d


def _prop_kernel(alpha_ref, ti_ref, tv_ref, d_ref, pb_ref, ps_ref, out_ref):
    col = jax.lax.broadcasted_iota(jnp.int32, (_RB, _B), 1)
    a = jnp.zeros((_RB, _B), jnp.float32)
    for t in range(_K):
        a = a + jnp.where(col == ti_ref[:, t][:, None],
                          tv_ref[:, t][:, None], 0.0)
    sp = jax.lax.dot_general(
        a, ps_ref[...], (((1,), (0,)), ((), ())),
        preferred_element_type=jnp.float32)
    alpha = alpha_ref[0, 0]
    d = d_ref[...]
    out_ref[...] = ((1.0 - alpha) + alpha * d * d) * pb_ref[...] \
        + (alpha * d) * sp


@functools.partial(jax.jit, static_argnames=("interpret",))
def kernel(features, preds, current_alpha, interpret=False):
    fn = pl.pallas_call(
        _normalize_kernel,
        out_shape=jax.ShapeDtypeStruct((_B, _D), jnp.float32),
        interpret=interpret,
    )(features)

    tv, ti = pl.pallas_call(
        _simtopk_kernel,
        grid=(_B // _RB,),
        in_specs=[
            pl.BlockSpec((_RB, _D), lambda i: (i, 0)),
            pl.BlockSpec((_B, _D), lambda i: (0, 0)),
        ],
        out_specs=[
            pl.BlockSpec((_RB, _K), lambda i: (i, 0)),
            pl.BlockSpec((_RB, _K), lambda i: (i, 0)),
        ],
        out_shape=[
            jax.ShapeDtypeStruct((_B, _K), jnp.float32),
            jax.ShapeDtypeStruct((_B, _K), jnp.int32),
        ],
        interpret=interpret,
    )(fn, fn)

    d, ps = pl.pallas_call(
        _scale_kernel,
        out_shape=[
            jax.ShapeDtypeStruct((_B, 1), jnp.float32),
            jax.ShapeDtypeStruct((_B, _C), jnp.float32),
        ],
        interpret=interpret,
    )(tv, preds)

    alpha = jnp.asarray(current_alpha, jnp.float32).reshape(1, 1)
    out = pl.pallas_call(
        _prop_kernel,
        grid=(_B // _RB,),
        in_specs=[
            pl.BlockSpec((1, 1), lambda i: (0, 0)),
            pl.BlockSpec((_RB, _K), lambda i: (i, 0)),
            pl.BlockSpec((_RB, _K), lambda i: (i, 0)),
            pl.BlockSpec((_RB, 1), lambda i: (i, 0)),
            pl.BlockSpec((_RB, _C), lambda i: (i, 0)),
            pl.BlockSpec((_B, _C), lambda i: (0, 0)),
        ],
        out_specs=pl.BlockSpec((_RB, _C), lambda i: (i, 0)),
        out_shape=jax.ShapeDtypeStruct((_B, _C), jnp.float32),
        interpret=interpret,
    )(alpha, ti, tv, d, preds, ps)
    return out


# speedup vs baseline: 9.9195x; 1.4641x over previous
import functools

import jax
import jax.numpy as jnp
from jax.experimental import pallas as pl

_B = 4096
_D = 1024
_C = 1000
_K = 32
_RB = 512


def _normalize_kernel(f_ref, out_ref):
    f = f_ref[...]
    n2 = jnp.sum(f * f, axis=1, keepdims=True)
    out_ref[...] = f * jax.lax.rsqrt(jnp.maximum(n2, 1e-24))


def _simtopk_kernel(fb_ref, fn_ref, tv_ref, ti_ref):
    sim = jax.lax.dot_general(
        fb_ref[...], fn_ref[...], (((1,), (1,)), ((), ())),
        preferred_element_type=jnp.float32)
    col = jax.lax.broadcasted_iota(jnp.int32, sim.shape, 1)
    vals = sim
    neg = jnp.float32(-3.0e38)
    for t in range(_K):
        m = jnp.max(vals, axis=1, keepdims=True)
        am = jnp.min(jnp.where(vals == m, col, _B), axis=1)
        tv_ref[:, t] = m[:, 0]
        ti_ref[:, t] = am
        vals = jnp.where(col == am[:, None], neg, vals)


def _scale_kernel(tv_ref, preds_ref, d_ref, ps_ref):
    rowsum = jnp.sum(tv_ref[...], axis=1, keepdims=True) + 1.0
    d = jax.lax.rsqrt(rowsum)
    d = jnp.where(jnp.isinf(d), 0.0, d)
    d_ref[...] = d
    ps_ref[...] = preds_ref[...] * d


def _prop_kernel(alpha_ref, ti_ref, tv_ref, d_ref, pb_ref, ps_ref, out_ref):
    col = jax.lax.broadcasted_iota(jnp.int32, (_RB, _B), 1)
    a = jnp.zeros((_RB, _B), jnp.float32)
    for t in range(_K):
        a = a + jnp.where(col == ti_ref[:, t][:, None],
                          tv_ref[:, t][:, None], 0.0)
    sp = jax.lax.dot_general(
        a, ps_ref[...], (((1,), (0,)), ((), ())),
        preferred_element_type=jnp.float32)
    alpha = alpha_ref[0, 0]
    d = d_ref[...]
    out_ref[...] = ((1.0 - alpha) + alpha * d * d) * pb_ref[...] \
        + (alpha * d) * sp


@functools.partial(jax.jit, static_argnames=("interpret",))
def kernel(features, preds, current_alpha, interpret=False):
    fn = pl.pallas_call(
        _normalize_kernel,
        out_shape=jax.ShapeDtypeStruct((_B, _D), jnp.float32),
        interpret=interpret,
    )(features)

    tv, ti = pl.pallas_call(
        _simtopk_kernel,
        grid=(_B // _RB,),
        in_specs=[
            pl.BlockSpec((_RB, _D), lambda i: (i, 0)),
            pl.BlockSpec((_B, _D), lambda i: (0, 0)),
        ],
        out_specs=[
            pl.BlockSpec((_RB, _K), lambda i: (i, 0)),
            pl.BlockSpec((_RB, _K), lambda i: (i, 0)),
        ],
        out_shape=[
            jax.ShapeDtypeStruct((_B, _K), jnp.float32),
            jax.ShapeDtypeStruct((_B, _K), jnp.int32),
        ],
        interpret=interpret,
    )(fn, fn)

    d, ps = pl.pallas_call(
        _scale_kernel,
        out_shape=[
            jax.ShapeDtypeStruct((_B, 1), jnp.float32),
            jax.ShapeDtypeStruct((_B, _C), jnp.float32),
        ],
        interpret=interpret,
    )(tv, preds)

    return tv, ti
    alpha = jnp.asarray(current_alpha, jnp.float32).reshape(1, 1)
    out = pl.pallas_call(
        _prop_kernel,
        grid=(_B // _RB,),
        in_specs=[
            pl.BlockSpec((1, 1), lambda i: (0, 0)),
            pl.BlockSpec((_RB, _K), lambda i: (i, 0)),
            pl.BlockSpec((_RB, _K), lambda i: (i, 0)),
            pl.BlockSpec((_RB, 1), lambda i: (i, 0)),
            pl.BlockSpec((_RB, _C), lambda i: (i, 0)),
            pl.BlockSpec((_B, _C), lambda i: (0, 0)),
        ],
        out_specs=pl.BlockSpec((_RB, _C), lambda i: (i, 0)),
        out_shape=jax.ShapeDtypeStruct((_B, _C), jnp.float32),
        interpret=interpret,
    )(alpha, ti, tv, d, preds, ps)
    return out


# speedup vs baseline: 10.0337x; 1.0115x over previous
import functools

import jax
import jax.numpy as jnp
from jax.experimental import pallas as pl

_B = 4096
_D = 1024
_C = 1000
_K = 32
_RB = 512
_IDX_BITS = 12
_IDX_MASK = (1 << _IDX_BITS) - 1


def _normalize_kernel(f_ref, out_ref):
    f = f_ref[...]
    n2 = jnp.sum(f * f, axis=1, keepdims=True)
    out_ref[...] = f * jax.lax.rsqrt(jnp.maximum(n2, 1e-24))


def _simtopk_kernel(fb_ref, fn_ref, tv_ref, ti_ref):
    sim = jax.lax.dot_general(
        fb_ref[...], fn_ref[...], (((1,), (1,)), ((), ())),
        preferred_element_type=jnp.float32)
    col = jax.lax.broadcasted_iota(jnp.int32, sim.shape, 1)
    u = jax.lax.bitcast_convert_type(sim, jnp.int32)
    key = jnp.where(u < 0, u ^ jnp.int32(0x7FFFFFFF), u)
    keyp = (key & jnp.int32(~_IDX_MASK)) | (_IDX_MASK - col)
    sentinel = jnp.int32(-2147483648)
    for t in range(_K):
        m = jnp.max(keyp, axis=1)
        ti_ref[:, t] = _IDX_MASK - (m & _IDX_MASK)
        kv = (m & jnp.int32(~_IDX_MASK)) | jnp.int32(0x800)
        uv = jnp.where(kv < 0, kv ^ jnp.int32(0x7FFFFFFF), kv)
        tv_ref[:, t] = jax.lax.bitcast_convert_type(uv, jnp.float32)
        keyp = jnp.where(keyp == m[:, None], sentinel, keyp)


def _scale_kernel(tv_ref, preds_ref, d_ref, ps_ref):
    rowsum = jnp.sum(tv_ref[...], axis=1, keepdims=True) + 1.0
    d = jax.lax.rsqrt(rowsum)
    d = jnp.where(jnp.isinf(d), 0.0, d)
    d_ref[...] = d
    ps_ref[...] = (preds_ref[...] * d).astype(jnp.bfloat16)


def _prop_kernel(alpha_ref, ti_ref, tv_ref, d_ref, pb_ref, ps_ref, out_ref):
    col = jax.lax.broadcasted_iota(jnp.int16, (_RB, _B), 1)
    ti16 = ti_ref[...].astype(jnp.int16)
    tvb = tv_ref[...].astype(jnp.bfloat16)
    a = jnp.zeros((_RB, _B), jnp.bfloat16)
    zero = jnp.bfloat16(0.0)
    for t in range(_K):
        a = a + jnp.where(col == ti16[:, t][:, None], tvb[:, t][:, None], zero)
    sp = jax.lax.dot_general(
        a, ps_ref[...], (((1,), (0,)), ((), ())),
        preferred_element_type=jnp.float32)
    alpha = alpha_ref[0, 0]
    d = d_ref[...]
    out_ref[...] = ((1.0 - alpha) + alpha * d * d) * pb_ref[...] \
        + (alpha * d) * sp


@functools.partial(jax.jit, static_argnames=("interpret",))
def kernel(features, preds, current_alpha, interpret=False):
    fn = pl.pallas_call(
        _normalize_kernel,
        out_shape=jax.ShapeDtypeStruct((_B, _D), jnp.float32),
        interpret=interpret,
    )(features)

    tv, ti = pl.pallas_call(
        _simtopk_kernel,
        grid=(_B // _RB,),
        in_specs=[
            pl.BlockSpec((_RB, _D), lambda i: (i, 0)),
            pl.BlockSpec((_B, _D), lambda i: (0, 0)),
        ],
        out_specs=[
            pl.BlockSpec((_RB, _K), lambda i: (i, 0)),
            pl.BlockSpec((_RB, _K), lambda i: (i, 0)),
        ],
        out_shape=[
            jax.ShapeDtypeStruct((_B, _K), jnp.float32),
            jax.ShapeDtypeStruct((_B, _K), jnp.int32),
        ],
        interpret=interpret,
    )(fn, fn)

    d, ps = pl.pallas_call(
        _scale_kernel,
        out_shape=[
            jax.ShapeDtypeStruct((_B, 1), jnp.float32),
            jax.ShapeDtypeStruct((_B, _C), jnp.bfloat16),
        ],
        interpret=interpret,
    )(tv, preds)

    alpha = jnp.asarray(current_alpha, jnp.float32).reshape(1, 1)
    out = pl.pallas_call(
        _prop_kernel,
        grid=(_B // _RB,),
        in_specs=[
            pl.BlockSpec((1, 1), lambda i: (0, 0)),
            pl.BlockSpec((_RB, _K), lambda i: (i, 0)),
            pl.BlockSpec((_RB, _K), lambda i: (i, 0)),
            pl.BlockSpec((_RB, 1), lambda i: (i, 0)),
            pl.BlockSpec((_RB, _C), lambda i: (i, 0)),
            pl.BlockSpec((_B, _C), lambda i: (0, 0)),
        ],
        out_specs=pl.BlockSpec((_RB, _C), lambda i: (i, 0)),
        out_shape=jax.ShapeDtypeStruct((_B, _C), jnp.float32),
        interpret=interpret,
    )(alpha, ti, tv, d, preds, ps)
    return out


# speedup vs baseline: 10.1197x; 1.0086x over previous
import functools

import jax
import jax.numpy as jnp
from jax.experimental import pallas as pl

_B = 4096
_D = 1024
_C = 1000
_K = 32
_RB = 512
_IDX_BITS = 12
_IDX_MASK = (1 << _IDX_BITS) - 1


def _normalize_kernel(f_ref, out_ref):
    f = f_ref[...]
    n2 = jnp.sum(f * f, axis=1, keepdims=True)
    out_ref[...] = f * jax.lax.rsqrt(jnp.maximum(n2, 1e-24))


def _simtopk_kernel(fb_ref, fn_ref, tv_ref, ti_ref):
    sim = jax.lax.dot_general(
        fb_ref[...], fn_ref[...], (((1,), (1,)), ((), ())),
        preferred_element_type=jnp.float32)
    col = jax.lax.broadcasted_iota(jnp.int32, sim.shape, 1)
    u = jax.lax.bitcast_convert_type(sim, jnp.int32)
    key = jnp.where(u < 0, u ^ jnp.int32(0x7FFFFFFF), u)
    keyp = (key & jnp.int32(~_IDX_MASK)) | (_IDX_MASK - col)
    sentinel = jnp.int32(-2147483648)
    m = jnp.max(keyp, axis=1)
    for t in range(_K):
        if t > 0:
            m = jnp.max(jnp.where(keyp < m[:, None], keyp, sentinel), axis=1)
        ti_ref[:, t] = _IDX_MASK - (m & _IDX_MASK)
        kv = (m & jnp.int32(~_IDX_MASK)) | jnp.int32(0x800)
        uv = jnp.where(kv < 0, kv ^ jnp.int32(0x7FFFFFFF), kv)
        tv_ref[:, t] = jax.lax.bitcast_convert_type(uv, jnp.float32)


def _scale_kernel(tv_ref, preds_ref, d_ref, ps_ref):
    rowsum = jnp.sum(tv_ref[...], axis=1, keepdims=True) + 1.0
    d = jax.lax.rsqrt(rowsum)
    d = jnp.where(jnp.isinf(d), 0.0, d)
    d_ref[...] = d
    ps_ref[...] = (preds_ref[...] * d).astype(jnp.bfloat16)


def _prop_kernel(alpha_ref, ti_ref, tv_ref, d_ref, pb_ref, ps_ref, out_ref):
    col = jax.lax.broadcasted_iota(jnp.int16, (_RB, _B), 1)
    ti16 = ti_ref[...].astype(jnp.int16)
    tvb = tv_ref[...].astype(jnp.bfloat16)
    a = jnp.zeros((_RB, _B), jnp.bfloat16)
    zero = jnp.bfloat16(0.0)
    for t in range(_K):
        a = a + jnp.where(col == ti16[:, t][:, None], tvb[:, t][:, None], zero)
    sp = jax.lax.dot_general(
        a, ps_ref[...], (((1,), (0,)), ((), ())),
        preferred_element_type=jnp.float32)
    alpha = alpha_ref[0, 0]
    d = d_ref[...]
    out_ref[...] = ((1.0 - alpha) + alpha * d * d) * pb_ref[...] \
        + (alpha * d) * sp


@functools.partial(jax.jit, static_argnames=("interpret",))
def kernel(features, preds, current_alpha, interpret=False):
    fn = pl.pallas_call(
        _normalize_kernel,
        out_shape=jax.ShapeDtypeStruct((_B, _D), jnp.float32),
        interpret=interpret,
    )(features)

    tv, ti = pl.pallas_call(
        _simtopk_kernel,
        grid=(_B // _RB,),
        in_specs=[
            pl.BlockSpec((_RB, _D), lambda i: (i, 0)),
            pl.BlockSpec((_B, _D), lambda i: (0, 0)),
        ],
        out_specs=[
            pl.BlockSpec((_RB, _K), lambda i: (i, 0)),
            pl.BlockSpec((_RB, _K), lambda i: (i, 0)),
        ],
        out_shape=[
            jax.ShapeDtypeStruct((_B, _K), jnp.float32),
            jax.ShapeDtypeStruct((_B, _K), jnp.int32),
        ],
        interpret=interpret,
    )(fn, fn)

    d, ps = pl.pallas_call(
        _scale_kernel,
        out_shape=[
            jax.ShapeDtypeStruct((_B, 1), jnp.float32),
            jax.ShapeDtypeStruct((_B, _C), jnp.bfloat16),
        ],
        interpret=interpret,
    )(tv, preds)

    alpha = jnp.asarray(current_alpha, jnp.float32).reshape(1, 1)
    out = pl.pallas_call(
        _prop_kernel,
        grid=(_B // _RB,),
        in_specs=[
            pl.BlockSpec((1, 1), lambda i: (0, 0)),
            pl.BlockSpec((_RB, _K), lambda i: (i, 0)),
            pl.BlockSpec((_RB, _K), lambda i: (i, 0)),
            pl.BlockSpec((_RB, 1), lambda i: (i, 0)),
            pl.BlockSpec((_RB, _C), lambda i: (i, 0)),
            pl.BlockSpec((_B, _C), lambda i: (0, 0)),
        ],
        out_specs=pl.BlockSpec((_RB, _C), lambda i: (i, 0)),
        out_shape=jax.ShapeDtypeStruct((_B, _C), jnp.float32),
        interpret=interpret,
    )(alpha, ti, tv, d, preds, ps)
    return out


# speedup vs baseline: 16.2627x; 1.6070x over previous
import functools

import jax
import jax.numpy as jnp
from jax.experimental import pallas as pl

_B = 4096
_D = 1024
_C = 1000
_K = 32
_RB = 512
_IDX_BITS = 12
_IDX_MASK = (1 << _IDX_BITS) - 1


def _normalize_kernel(f_ref, out_ref):
    f = f_ref[...]
    n2 = jnp.sum(f * f, axis=1, keepdims=True)
    out_ref[...] = f * jax.lax.rsqrt(jnp.maximum(n2, 1e-24))


def _simselect_kernel(fb_ref, fn_ref, a_ref, rs_ref):
    sim = jax.lax.dot_general(
        fb_ref[...], fn_ref[...], (((1,), (1,)), ((), ())),
        preferred_element_type=jnp.float32)
    col = jax.lax.broadcasted_iota(jnp.int32, sim.shape, 1)
    u = jax.lax.bitcast_convert_type(sim, jnp.int32)
    key = jnp.where(u < 0, u ^ jnp.int32(0x7FFFFFFF), u)
    keyp = (key & jnp.int32(~_IDX_MASK)) | (_IDX_MASK - col)
    cnt0 = jnp.sum((keyp >= 0).astype(jnp.int32), axis=1, keepdims=True)
    t = jnp.where(cnt0 >= _K, jnp.int32(0), jnp.int32(-2147483648))
    for b in range(30, -1, -1):
        tc = t | jnp.int32(1 << b)
        cnt = jnp.sum((keyp >= tc).astype(jnp.int32), axis=1, keepdims=True)
        t = jnp.where(cnt >= _K, tc, t)
    a_f = jnp.where(keyp >= t, sim, 0.0)
    rs_ref[...] = jnp.sum(a_f, axis=1, keepdims=True)
    a_ref[...] = a_f.astype(jnp.bfloat16)


def _scale_kernel(rs_ref, preds_ref, d_ref, ps_ref):
    d = jax.lax.rsqrt(rs_ref[...] + 1.0)
    d = jnp.where(jnp.isinf(d), 0.0, d)
    d_ref[...] = d
    ps_ref[...] = (preds_ref[...] * d).astype(jnp.bfloat16)


def _prop_kernel(alpha_ref, a_ref, d_ref, pb_ref, ps_ref, out_ref):
    sp = jax.lax.dot_general(
        a_ref[...], ps_ref[...], (((1,), (0,)), ((), ())),
        preferred_element_type=jnp.float32)
    alpha = alpha_ref[0, 0]
    d = d_ref[...]
    out_ref[...] = ((1.0 - alpha) + alpha * d * d) * pb_ref[...] \
        + (alpha * d) * sp


@functools.partial(jax.jit, static_argnames=("interpret",))
def kernel(features, preds, current_alpha, interpret=False):
    fn = pl.pallas_call(
        _normalize_kernel,
        out_shape=jax.ShapeDtypeStruct((_B, _D), jnp.float32),
        interpret=interpret,
    )(features)

    adj, rs = pl.pallas_call(
        _simselect_kernel,
        grid=(_B // _RB,),
        in_specs=[
            pl.BlockSpec((_RB, _D), lambda i: (i, 0)),
            pl.BlockSpec((_B, _D), lambda i: (0, 0)),
        ],
        out_specs=[
            pl.BlockSpec((_RB, _B), lambda i: (i, 0)),
            pl.BlockSpec((_RB, 1), lambda i: (i, 0)),
        ],
        out_shape=[
            jax.ShapeDtypeStruct((_B, _B), jnp.bfloat16),
            jax.ShapeDtypeStruct((_B, 1), jnp.float32),
        ],
        interpret=interpret,
    )(fn, fn)

    d, ps = pl.pallas_call(
        _scale_kernel,
        out_shape=[
            jax.ShapeDtypeStruct((_B, 1), jnp.float32),
            jax.ShapeDtypeStruct((_B, _C), jnp.bfloat16),
        ],
        interpret=interpret,
    )(rs, preds)

    alpha = jnp.asarray(current_alpha, jnp.float32).reshape(1, 1)
    out = pl.pallas_call(
        _prop_kernel,
        grid=(_B // _RB,),
        in_specs=[
            pl.BlockSpec((1, 1), lambda i: (0, 0)),
            pl.BlockSpec((_RB, _B), lambda i: (i, 0)),
            pl.BlockSpec((_RB, 1), lambda i: (i, 0)),
            pl.BlockSpec((_RB, _C), lambda i: (i, 0)),
            pl.BlockSpec((_B, _C), lambda i: (0, 0)),
        ],
        out_specs=pl.BlockSpec((_RB, _C), lambda i: (i, 0)),
        out_shape=jax.ShapeDtypeStruct((_B, _C), jnp.float32),
        interpret=interpret,
    )(alpha, adj, d, preds, ps)
    return out


# speedup vs baseline: 17.2625x; 1.0615x over previous
import functools

import jax
import jax.numpy as jnp
from jax.experimental import pallas as pl

_B = 4096
_D = 1024
_C = 1000
_K = 32
_RB = 512
_IDX_BITS = 12
_IDX_MASK = (1 << _IDX_BITS) - 1


def _normalize_kernel(f_ref, out_ref):
    f = f_ref[...]
    n2 = jnp.sum(f * f, axis=1, keepdims=True)
    out_ref[...] = f * jax.lax.rsqrt(jnp.maximum(n2, 1e-24))


def _simselect_kernel(fb_ref, fn_ref, a_ref, rs_ref):
    sim = jax.lax.dot_general(
        fb_ref[...], fn_ref[...], (((1,), (1,)), ((), ())),
        preferred_element_type=jnp.float32)
    col = jax.lax.broadcasted_iota(jnp.int32, sim.shape, 1)
    u = jax.lax.bitcast_convert_type(sim, jnp.int32)
    key = jnp.where(u < 0, u ^ jnp.int32(0x7FFFFFFF), u)
    keyp = (key & jnp.int32(~_IDX_MASK)) | (_IDX_MASK - col)
    khi = (keyp >> 16).astype(jnp.int16)
    klo = (keyp & jnp.int32(0xFFFF)).astype(jnp.int16) ^ jnp.int16(-32768)
    one16 = jnp.int16(1)
    zero16 = jnp.int16(0)

    def count(mask):
        x = jnp.where(mask, one16, zero16)
        w = x.shape[1]
        while w > 256:
            w //= 2
            x = x[:, :w] + x[:, w:]
        return jnp.sum(x.astype(jnp.int32), axis=1,
                       keepdims=True).astype(jnp.int16)

    cnt0 = count(khi >= 0)
    k16 = jnp.int16(_K)
    thi = jnp.where(cnt0 >= k16, jnp.int16(0), jnp.int16(-32768))
    for b in range(14, -1, -1):
        tc = thi | jnp.int16(1 << b)
        thi = jnp.where(count(khi >= tc) >= k16, tc, thi)
    bnd = khi == thi
    need = k16 - count(khi > thi)
    cnt0l = count(bnd & (klo >= 0))
    tlo = jnp.where(cnt0l >= need, jnp.int16(0), jnp.int16(-32768))
    for b in range(14, -1, -1):
        tc = tlo | jnp.int16(1 << b)
        tlo = jnp.where(count(bnd & (klo >= tc)) >= need, tc, tlo)
    sel = (khi > thi) | (bnd & (klo >= tlo))
    a_f = jnp.where(sel, sim, 0.0)
    rs_ref[...] = jnp.sum(a_f, axis=1, keepdims=True)
    a_ref[...] = a_f.astype(jnp.bfloat16)


def _scale_kernel(rs_ref, preds_ref, d_ref, ps_ref):
    d = jax.lax.rsqrt(rs_ref[...] + 1.0)
    d = jnp.where(jnp.isinf(d), 0.0, d)
    d_ref[...] = d
    ps_ref[...] = (preds_ref[...] * d).astype(jnp.bfloat16)


def _prop_kernel(alpha_ref, a_ref, d_ref, pb_ref, ps_ref, out_ref):
    sp = jax.lax.dot_general(
        a_ref[...], ps_ref[...], (((1,), (0,)), ((), ())),
        preferred_element_type=jnp.float32)
    alpha = alpha_ref[0, 0]
    d = d_ref[...]
    out_ref[...] = ((1.0 - alpha) + alpha * d * d) * pb_ref[...] \
        + (alpha * d) * sp


@functools.partial(jax.jit, static_argnames=("interpret",))
def kernel(features, preds, current_alpha, interpret=False):
    fn = pl.pallas_call(
        _normalize_kernel,
        out_shape=jax.ShapeDtypeStruct((_B, _D), jnp.float32),
        interpret=interpret,
    )(features)

    adj, rs = pl.pallas_call(
        _simselect_kernel,
        grid=(_B // _RB,),
        in_specs=[
            pl.BlockSpec((_RB, _D), lambda i: (i, 0)),
            pl.BlockSpec((_B, _D), lambda i: (0, 0)),
        ],
        out_specs=[
            pl.BlockSpec((_RB, _B), lambda i: (i, 0)),
            pl.BlockSpec((_RB, 1), lambda i: (i, 0)),
        ],
        out_shape=[
            jax.ShapeDtypeStruct((_B, _B), jnp.bfloat16),
            jax.ShapeDtypeStruct((_B, 1), jnp.float32),
        ],
        interpret=interpret,
    )(fn, fn)

    d, ps = pl.pallas_call(
        _scale_kernel,
        out_shape=[
            jax.ShapeDtypeStruct((_B, 1), jnp.float32),
            jax.ShapeDtypeStruct((_B, _C), jnp.bfloat16),
        ],
        interpret=interpret,
    )(rs, preds)

    alpha = jnp.asarray(current_alpha, jnp.float32).reshape(1, 1)
    out = pl.pallas_call(
        _prop_kernel,
        grid=(_B // _RB,),
        in_specs=[
            pl.BlockSpec((1, 1), lambda i: (0, 0)),
            pl.BlockSpec((_RB, _B), lambda i: (i, 0)),
            pl.BlockSpec((_RB, 1), lambda i: (i, 0)),
            pl.BlockSpec((_RB, _C), lambda i: (i, 0)),
            pl.BlockSpec((_B, _C), lambda i: (0, 0)),
        ],
        out_specs=pl.BlockSpec((_RB, _C), lambda i: (i, 0)),
        out_shape=jax.ShapeDtypeStruct((_B, _C), jnp.float32),
        interpret=interpret,
    )(alpha, adj, d, preds, ps)
    return out
